# baseline (speedup 1.0000x reference)
import functools

import jax
import jax.numpy as jnp
from jax import lax
from jax.experimental import pallas as pl
from jax.experimental.pallas import tpu as pltpu
from jax.experimental.pallas import tpu_sc as plsc

H, W, C = 1024, 1024, 2
T, N1 = 200, 16384

NC = 2
NS = 16
L = 16
NW = NC * NS

RPW = H // NW
RCH = 8
NCH = RPW // RCH

B = 2048
NCK = T * (N1 // B)
CPW = NCK // NW
NSTR = B // 128
NG = B // L
TPL = 2 * N1


def _build_body(grid_hbm, grid8_hbm, rbuf, obuf):
    ids = lax.iota(jnp.int32, L)
    wid = lax.axis_index("s") * NC + lax.axis_index("c")
    row0 = wid * RPW

    def chunk(ck, carry):
        crow = row0 + ck * RCH
        start = jnp.minimum(crow, H - (RCH + 1))
        off = crow - start
        pltpu.sync_copy(grid_hbm.at[pl.ds(start * 2 * W, (RCH + 1) * 2 * W)],
                        rbuf.at[pl.ds(0, (RCH + 1) * 2 * W)])

        def group(g, _):
            ri = g // (W // L)
            gx = g - ri * (W // L)
            x = gx * L + ids
            ro = (ri + off) * (2 * W)
            s00_0 = plsc.load_gather(rbuf, [ro + x])
            s00_1 = plsc.load_gather(rbuf, [ro + W + x])
            s01_0 = plsc.load_gather(rbuf, [ro + x + 1])
            s01_1 = plsc.load_gather(rbuf, [ro + W + x + 1])
            s10_0 = plsc.load_gather(rbuf, [ro + 2 * W + x])
            s10_1 = plsc.load_gather(rbuf, [ro + 3 * W + x])
            s11_0 = plsc.load_gather(rbuf, [ro + 2 * W + x + 1])
            s11_1 = plsc.load_gather(rbuf, [ro + 3 * W + x + 1])
            pk = plsc.PackFormat.INTERLEAVED
            w00 = plsc.bitcast(plsc.pack(s00_0, s00_1, format=pk), jnp.float32)
            w01 = plsc.bitcast(plsc.pack(s01_0, s01_1, format=pk), jnp.float32)
            w10 = plsc.bitcast(plsc.pack(s10_0, s10_1, format=pk), jnp.float32)
            w11 = plsc.bitcast(plsc.pack(s11_0, s11_1, format=pk), jnp.float32)
            o4 = x * 4 + ri * (4 * W)
            plsc.store_scatter(obuf, [o4], w00)
            plsc.store_scatter(obuf, [o4 + 1], w01)
            plsc.store_scatter(obuf, [o4 + 2], w10)
            plsc.store_scatter(obuf, [o4 + 3], w11)
            return _

        lax.fori_loop(0, RCH * (W // L), group, 0)
        pltpu.sync_copy(obuf, grid8_hbm.at[pl.ds(crow * 4 * W, RCH * 4 * W)])
        return carry

    lax.fori_loop(0, NCH, chunk, 0)


def _build_grid8(grid_planar):
    mesh = plsc.VectorSubcoreMesh(core_axis_name="c", subcore_axis_name="s")
    return pl.kernel(
        _build_body,
        out_type=jax.ShapeDtypeStruct((H * W * 4,), jnp.float32),
        mesh=mesh,
        compiler_params=pltpu.CompilerParams(
            needs_layout_passes=False, use_tc_tiling_on_sc=False),
        scratch_types=[
            pltpu.VMEM(((RCH + 2) * 2 * W + 64,), jnp.float32),
            pltpu.VMEM((RCH * 4 * W,), jnp.float32),
        ],
    )(grid_planar)


NPT = N1 // B


def _sc_body(coords_hbm, grid8_hbm, out_hbm,
             xb0, xb1, yb0, yb1, ib0, ib1, wx0, wx1, wy0, wy1,
             gb0, gb1, ox0, ox1, oy0, oy1, csem, gsem, osem):
    ids = lax.iota(jnp.int32, L)
    wid = lax.axis_index("s") * NC + lax.axis_index("c")
    u0 = wid * CPW
    xb, yb, ib = (xb0, xb1), (yb0, yb1), (ib0, ib1)
    wxb, wyb = (wx0, wx1), (wy0, wy1)
    gb, oxb, oyb = (gb0, gb1), (ox0, ox1), (oy0, oy1)

    def addr(u):
        t = u // NPT
        return t * TPL + (u - t * NPT) * B

    def p1(xbuf, ybuf, ibuf, wxbuf, wybuf):
        def f(gi, carry):
            x = xbuf[pl.ds(gi * L, L)]
            y = ybuf[pl.ds(gi * L, L)]
            xs = x * jnp.float32(W - 1)
            ys = y * jnp.float32(H - 1)
            x0 = xs.astype(jnp.int32)
            y0 = ys.astype(jnp.int32)
            ibuf[pl.ds(gi * L, L)] = y0 * W + x0
            wxbuf[pl.ds(gi * L, L)] = xs - x0.astype(jnp.float32)
            wybuf[pl.ds(gi * L, L)] = ys - y0.astype(jnp.float32)
            return carry

        lax.fori_loop(0, NG, f, 0, unroll=4)

    def p3(gbuf, wxbuf, wybuf, oxbuf, oybuf):
        def f(gi, carry):
            bse = gi * L
            rows = ids + bse
            q = [plsc.load_gather(gbuf, [rows, jnp.full((L,), k, jnp.int32)])
                 for k in range(4)]
            pk = plsc.PackFormat.INTERLEAVED
            g00_0, g00_1 = plsc.unpack(plsc.bitcast(q[0], jnp.bfloat16), format=pk)
            g01_0, g01_1 = plsc.unpack(plsc.bitcast(q[1], jnp.bfloat16), format=pk)
            g10_0, g10_1 = plsc.unpack(plsc.bitcast(q[2], jnp.bfloat16), format=pk)
            g11_0, g11_1 = plsc.unpack(plsc.bitcast(q[3], jnp.bfloat16), format=pk)
            wx = wxbuf[pl.ds(bse, L)]
            wy = wybuf[pl.ds(bse, L)]
            top0 = g00_0 + wx * (g01_0 - g00_0)
            top1 = g00_1 + wx * (g01_1 - g00_1)
            bot0 = g10_0 + wx * (g11_0 - g10_0)
            bot1 = g10_1 + wx * (g11_1 - g10_1)
            oxbuf[pl.ds(bse, L)] = top0 + wy * (bot0 - top0)
            oybuf[pl.ds(bse, L)] = top1 + wy * (bot1 - top1)
            return carry

        lax.fori_loop(0, NG, f, 0, unroll=4)

    b0 = addr(u0)
    pltpu.async_copy(coords_hbm.at[pl.ds(b0, B)], xb[0], csem)
    pltpu.async_copy(coords_hbm.at[pl.ds(b0 + N1, B)], yb[0], csem)

    def iteration(k, cur):
        nxt = 1 - cur
        base = addr(u0 + k)

        @pl.when(k + 1 < CPW)
        def _():
            bn = addr(u0 + k + 1)
            pltpu.async_copy(coords_hbm.at[pl.ds(bn, B)], xb[nxt], csem)
            pltpu.async_copy(coords_hbm.at[pl.ds(bn + N1, B)], yb[nxt], csem)

        pltpu.make_async_copy(coords_hbm.at[pl.ds(base, B)], xb[cur], csem).wait()
        pltpu.make_async_copy(coords_hbm.at[pl.ds(base + N1, B)], yb[cur], csem).wait()

        p1(xb[cur], yb[cur], ib[cur], wxb[cur], wyb[cur])
        for s in range(NSTR):
            pltpu.async_copy(
                grid8_hbm.at[ib[cur].at[pl.ds(s * 128, 128)]],
                gb[cur].at[pl.ds(s * 128, 128)], gsem)

        @pl.when(k >= 1)
        def _():
            @pl.when(k >= 2)
            def _():
                bq = addr(u0 + k - 2)
                pltpu.make_async_copy(oxb[cur], out_hbm.at[pl.ds(bq, B)], osem).wait()
                pltpu.make_async_copy(oyb[cur], out_hbm.at[pl.ds(bq + N1, B)], osem).wait()
            p3(gb[nxt], wxb[nxt], wyb[nxt], oxb[nxt], oyb[nxt])
            bp = addr(u0 + k - 1)
            pltpu.async_copy(oxb[nxt], out_hbm.at[pl.ds(bp, B)], osem)
            pltpu.async_copy(oyb[nxt], out_hbm.at[pl.ds(bp + N1, B)], osem)

        for s in range(NSTR):
            pltpu.make_async_copy(
                grid8_hbm.at[ib[cur].at[pl.ds(s * 128, 128)]],
                gb[cur].at[pl.ds(s * 128, 128)], gsem).wait()

    def two(j2, carry):
        iteration(j2 * 2, 0)
        iteration(j2 * 2 + 1, 1)
        return carry

    lax.fori_loop(0, CPW // 2, two, 0)

    lastp = (CPW - 1) % 2
    p3(gb[lastp], wxb[lastp], wyb[lastp], oxb[lastp], oyb[lastp])
    bl = addr(u0 + CPW - 1)
    pltpu.async_copy(oxb[lastp], out_hbm.at[pl.ds(bl, B)], osem)
    pltpu.async_copy(oyb[lastp], out_hbm.at[pl.ds(bl + N1, B)], osem)
    for _i in range(4):
        pltpu.make_async_copy(oxb[0], out_hbm.at[pl.ds(bl, B)], osem).wait()


def _sample(coords_planar, grid8):
    mesh = plsc.VectorSubcoreMesh(core_axis_name="c", subcore_axis_name="s")
    return pl.kernel(
        _sc_body,
        out_type=jax.ShapeDtypeStruct((T * 2 * N1,), jnp.float32),
        mesh=mesh,
        compiler_params=pltpu.CompilerParams(
            needs_layout_passes=False, use_tc_tiling_on_sc=False),
        scratch_types=(
            [pltpu.VMEM((B,), jnp.float32)] * 4
            + [pltpu.VMEM((B,), jnp.int32)] * 2
            + [pltpu.VMEM((B,), jnp.float32)] * 4
            + [pltpu.VMEM((B, 4), jnp.float32)] * 2
            + [pltpu.VMEM((B,), jnp.float32)] * 4
            + [pltpu.SemaphoreType.DMA] * 3
        ),
    )(coords_planar, grid8)


def kernel(coords, vector_field):
    coords_planar = jnp.transpose(coords, (1, 2, 0)).reshape(-1)
    grid_planar = jnp.transpose(vector_field, (0, 2, 1)).reshape(-1)
    grid8 = _build_grid8(grid_planar).reshape(H * W, 4)
    out_flat = _sample(coords_planar, grid8)
    return jnp.transpose(out_flat.reshape(T, 2, N1), (2, 0, 1))


# speedup vs baseline: 4.2817x; 4.2817x over previous
import functools

import jax
import jax.numpy as jnp
from jax import lax
from jax.experimental import pallas as pl
from jax.experimental.pallas import tpu as pltpu
from jax.experimental.pallas import tpu_sc as plsc

H, W, C = 1024, 1024, 2
T, N1 = 200, 16384

NC = 2
NS = 16
L = 16
NW = NC * NS

RPW = H // NW
RCH = 8
NCH = RPW // RCH

B = 2048
NCK = T * (N1 // B)
CPW = NCK // NW
NSTR = B // 128
NG = B // L
TPL = 2 * N1


def _build_body(grid_hbm, grid8_hbm, rbuf, obuf):
    ids = lax.iota(jnp.int32, L)
    wid = lax.axis_index("s") * NC + lax.axis_index("c")
    row0 = wid * RPW

    def chunk(ck, carry):
        crow = row0 + ck * RCH
        start = jnp.minimum(crow, H - (RCH + 1))
        off = crow - start
        pltpu.sync_copy(grid_hbm.at[pl.ds(start * 2 * W, (RCH + 1) * 2 * W)],
                        rbuf.at[pl.ds(0, (RCH + 1) * 2 * W)])

        def group(g, _):
            ri = g // (W // L)
            gx = g - ri * (W // L)
            x = gx * L + ids
            ro = (ri + off) * (2 * W)
            s00_0 = plsc.load_gather(rbuf, [ro + x])
            s00_1 = plsc.load_gather(rbuf, [ro + W + x])
            s01_0 = plsc.load_gather(rbuf, [ro + x + 1])
            s01_1 = plsc.load_gather(rbuf, [ro + W + x + 1])
            s10_0 = plsc.load_gather(rbuf, [ro + 2 * W + x])
            s10_1 = plsc.load_gather(rbuf, [ro + 3 * W + x])
            s11_0 = plsc.load_gather(rbuf, [ro + 2 * W + x + 1])
            s11_1 = plsc.load_gather(rbuf, [ro + 3 * W + x + 1])
            o8 = x * 8 + ri * (8 * W)
            plsc.store_scatter(obuf, [o8], s00_0)
            plsc.store_scatter(obuf, [o8 + 1], s00_1)
            plsc.store_scatter(obuf, [o8 + 2], s01_0)
            plsc.store_scatter(obuf, [o8 + 3], s01_1)
            plsc.store_scatter(obuf, [o8 + 4], s10_0)
            plsc.store_scatter(obuf, [o8 + 5], s10_1)
            plsc.store_scatter(obuf, [o8 + 6], s11_0)
            plsc.store_scatter(obuf, [o8 + 7], s11_1)
            return _

        lax.fori_loop(0, RCH * (W // L), group, 0)
        pltpu.sync_copy(obuf, grid8_hbm.at[pl.ds(crow * 8 * W, RCH * 8 * W)])
        return carry

    lax.fori_loop(0, NCH, chunk, 0)


def _build_grid8(grid_planar):
    mesh = plsc.VectorSubcoreMesh(core_axis_name="c", subcore_axis_name="s")
    return pl.kernel(
        _build_body,
        out_type=jax.ShapeDtypeStruct((H * W * 8,), jnp.float32),
        mesh=mesh,
        compiler_params=pltpu.CompilerParams(
            needs_layout_passes=False, use_tc_tiling_on_sc=False),
        scratch_types=[
            pltpu.VMEM(((RCH + 2) * 2 * W + 64,), jnp.float32),
            pltpu.VMEM((RCH * 8 * W,), jnp.float32),
        ],
    )(grid_planar)


NPT = N1 // B


def _sc_body(coords_hbm, grid8_hbm, out_hbm,
             xb0, xb1, yb0, yb1, ib0, ib1, wx0, wx1, wy0, wy1,
             gb0, gb1, ox0, ox1, oy0, oy1, csem, gsem, osem):
    ids = lax.iota(jnp.int32, L)
    wid = lax.axis_index("s") * NC + lax.axis_index("c")
    u0 = wid * CPW
    xb, yb, ib = (xb0, xb1), (yb0, yb1), (ib0, ib1)
    wxb, wyb = (wx0, wx1), (wy0, wy1)
    gb, oxb, oyb = (gb0, gb1), (ox0, ox1), (oy0, oy1)

    def addr(u):
        t = u // NPT
        return t * TPL + (u - t * NPT) * B

    def p1(xbuf, ybuf, ibuf, wxbuf, wybuf):
        def f(gi, carry):
            x = xbuf[pl.ds(gi * L, L)]
            y = ybuf[pl.ds(gi * L, L)]
            xs = x * jnp.float32(W - 1)
            ys = y * jnp.float32(H - 1)
            x0 = xs.astype(jnp.int32)
            y0 = ys.astype(jnp.int32)
            ibuf[pl.ds(gi * L, L)] = y0 * W + x0
            wxbuf[pl.ds(gi * L, L)] = xs - x0.astype(jnp.float32)
            wybuf[pl.ds(gi * L, L)] = ys - y0.astype(jnp.float32)
            return carry

        lax.fori_loop(0, NG, f, 0, unroll=8)

    def p3(gbuf, wxbuf, wybuf, oxbuf, oybuf):
        def f(gi, carry):
            bse = gi * L
            rows = ids + bse
            gv = [plsc.load_gather(gbuf, [rows, jnp.full((L,), k, jnp.int32)])
                  for k in range(8)]
            wx = wxbuf[pl.ds(bse, L)]
            wy = wybuf[pl.ds(bse, L)]
            top0 = gv[0] + wx * (gv[2] - gv[0])
            top1 = gv[1] + wx * (gv[3] - gv[1])
            bot0 = gv[4] + wx * (gv[6] - gv[4])
            bot1 = gv[5] + wx * (gv[7] - gv[5])
            oxbuf[pl.ds(bse, L)] = top0 + wy * (bot0 - top0)
            oybuf[pl.ds(bse, L)] = top1 + wy * (bot1 - top1)
            return carry

        lax.fori_loop(0, NG, f, 0, unroll=8)

    b0 = addr(u0)
    pltpu.async_copy(coords_hbm.at[pl.ds(b0, B)], xb[0], csem)
    pltpu.async_copy(coords_hbm.at[pl.ds(b0 + N1, B)], yb[0], csem)

    def iteration(k, cur):
        nxt = 1 - cur
        base = addr(u0 + k)

        @pl.when(k + 1 < CPW)
        def _():
            bn = addr(u0 + k + 1)
            pltpu.async_copy(coords_hbm.at[pl.ds(bn, B)], xb[nxt], csem)
            pltpu.async_copy(coords_hbm.at[pl.ds(bn + N1, B)], yb[nxt], csem)

        pltpu.make_async_copy(coords_hbm.at[pl.ds(base, B)], xb[cur], csem).wait()
        pltpu.make_async_copy(coords_hbm.at[pl.ds(base + N1, B)], yb[cur], csem).wait()

        p1(xb[cur], yb[cur], ib[cur], wxb[cur], wyb[cur])
        for s in range(NSTR):
            pltpu.async_copy(
                grid8_hbm.at[ib[cur].at[pl.ds(s * 128, 128)]],
                gb[cur].at[pl.ds(s * 128, 128)], gsem)

        @pl.when(k >= 1)
        def _():
            @pl.when(k >= 2)
            def _():
                bq = addr(u0 + k - 2)
                pltpu.make_async_copy(oxb[cur], out_hbm.at[pl.ds(bq, B)], osem).wait()
                pltpu.make_async_copy(oyb[cur], out_hbm.at[pl.ds(bq + N1, B)], osem).wait()
            p3(gb[nxt], wxb[nxt], wyb[nxt], oxb[nxt], oyb[nxt])
            bp = addr(u0 + k - 1)
            pltpu.async_copy(oxb[nxt], out_hbm.at[pl.ds(bp, B)], osem)
            pltpu.async_copy(oyb[nxt], out_hbm.at[pl.ds(bp + N1, B)], osem)

        for s in range(NSTR):
            pltpu.make_async_copy(
                grid8_hbm.at[ib[cur].at[pl.ds(s * 128, 128)]],
                gb[cur].at[pl.ds(s * 128, 128)], gsem).wait()

    def two(j2, carry):
        iteration(j2 * 2, 0)
        iteration(j2 * 2 + 1, 1)
        return carry

    lax.fori_loop(0, CPW // 2, two, 0)

    lastp = (CPW - 1) % 2
    p3(gb[lastp], wxb[lastp], wyb[lastp], oxb[lastp], oyb[lastp])
    bl = addr(u0 + CPW - 1)
    pltpu.async_copy(oxb[lastp], out_hbm.at[pl.ds(bl, B)], osem)
    pltpu.async_copy(oyb[lastp], out_hbm.at[pl.ds(bl + N1, B)], osem)
    for _i in range(4):
        pltpu.make_async_copy(oxb[0], out_hbm.at[pl.ds(bl, B)], osem).wait()


def _sample(coords_planar, grid8):
    mesh = plsc.VectorSubcoreMesh(core_axis_name="c", subcore_axis_name="s")
    return pl.kernel(
        _sc_body,
        out_type=jax.ShapeDtypeStruct((T * 2 * N1,), jnp.float32),
        mesh=mesh,
        compiler_params=pltpu.CompilerParams(
            needs_layout_passes=False, use_tc_tiling_on_sc=False),
        scratch_types=(
            [pltpu.VMEM((B,), jnp.float32)] * 4
            + [pltpu.VMEM((B,), jnp.int32)] * 2
            + [pltpu.VMEM((B,), jnp.float32)] * 4
            + [pltpu.VMEM((B, 8), jnp.float32)] * 2
            + [pltpu.VMEM((B,), jnp.float32)] * 4
            + [pltpu.SemaphoreType.DMA] * 3
        ),
    )(coords_planar, grid8)


def kernel(coords, vector_field):
    coords_planar = jnp.transpose(coords, (1, 2, 0)).reshape(-1)
    grid_planar = jnp.transpose(vector_field, (0, 2, 1)).reshape(-1)
    grid8 = _build_grid8(grid_planar).reshape(H * W, 8)
    out_flat = _sample(coords_planar, grid8)
    return jnp.transpose(out_flat.reshape(T, 2, N1), (2, 0, 1))


# speedup vs baseline: 4.3178x; 1.0084x over previous
import functools

import jax
import jax.numpy as jnp
from jax import lax
from jax.experimental import pallas as pl
from jax.experimental.pallas import tpu as pltpu
from jax.experimental.pallas import tpu_sc as plsc

H, W, C = 1024, 1024, 2
T, N1 = 200, 16384

NC = 2
NS = 16
L = 16
NW = NC * NS

RPW = H // NW
RCH = 8
NCH = RPW // RCH

B = 2048
NCK = T * (N1 // B)
CPW = NCK // NW
NSTR = B // 128
NG = B // L
TPL = 2 * N1


def _build_body(grid_hbm, grid8_hbm, rbuf, obuf):
    ids = lax.iota(jnp.int32, L)
    wid = lax.axis_index("s") * NC + lax.axis_index("c")
    row0 = wid * RPW

    def chunk(ck, carry):
        crow = row0 + ck * RCH
        start = jnp.minimum(crow, H - (RCH + 1))
        off = crow - start
        pltpu.sync_copy(grid_hbm.at[pl.ds(start * 2 * W, (RCH + 1) * 2 * W)],
                        rbuf.at[pl.ds(0, (RCH + 1) * 2 * W)])

        def group(g, _):
            ri = g // (W // L)
            gx = g - ri * (W // L)
            x = gx * L + ids
            ro = (ri + off) * (2 * W)
            s00_0 = plsc.load_gather(rbuf, [ro + x])
            s00_1 = plsc.load_gather(rbuf, [ro + W + x])
            s01_0 = plsc.load_gather(rbuf, [ro + x + 1])
            s01_1 = plsc.load_gather(rbuf, [ro + W + x + 1])
            s10_0 = plsc.load_gather(rbuf, [ro + 2 * W + x])
            s10_1 = plsc.load_gather(rbuf, [ro + 3 * W + x])
            s11_0 = plsc.load_gather(rbuf, [ro + 2 * W + x + 1])
            s11_1 = plsc.load_gather(rbuf, [ro + 3 * W + x + 1])
            o8 = x * 8 + ri * (8 * W)
            plsc.store_scatter(obuf, [o8], s00_0)
            plsc.store_scatter(obuf, [o8 + 1], s00_1)
            plsc.store_scatter(obuf, [o8 + 2], s01_0)
            plsc.store_scatter(obuf, [o8 + 3], s01_1)
            plsc.store_scatter(obuf, [o8 + 4], s10_0)
            plsc.store_scatter(obuf, [o8 + 5], s10_1)
            plsc.store_scatter(obuf, [o8 + 6], s11_0)
            plsc.store_scatter(obuf, [o8 + 7], s11_1)
            return _

        lax.fori_loop(0, RCH * (W // L), group, 0)
        pltpu.sync_copy(obuf, grid8_hbm.at[pl.ds(crow * 8 * W, RCH * 8 * W)])
        return carry

    lax.fori_loop(0, NCH, chunk, 0)


def _build_grid8(grid_planar):
    mesh = plsc.VectorSubcoreMesh(core_axis_name="c", subcore_axis_name="s")
    return pl.kernel(
        _build_body,
        out_type=jax.ShapeDtypeStruct((H * W * 8,), jnp.float32),
        mesh=mesh,
        compiler_params=pltpu.CompilerParams(
            needs_layout_passes=False, use_tc_tiling_on_sc=False),
        scratch_types=[
            pltpu.VMEM(((RCH + 2) * 2 * W + 64,), jnp.float32),
            pltpu.VMEM((RCH * 8 * W,), jnp.float32),
        ],
    )(grid_planar)


NPT = N1 // B


def _sc_body(coords_hbm, grid8_hbm, out_hbm,
             xb0, xb1, yb0, yb1, ib0, ib1, wx0, wx1, wy0, wy1,
             gb0, gb1, ox0, ox1, oy0, oy1, csem, gsem, osem):
    ids = lax.iota(jnp.int32, L)
    wid = lax.axis_index("s") * NC + lax.axis_index("c")
    u0 = wid * CPW
    xb, yb, ib = (xb0, xb1), (yb0, yb1), (ib0, ib1)
    wxb, wyb = (wx0, wx1), (wy0, wy1)
    gb, oxb, oyb = (gb0, gb1), (ox0, ox1), (oy0, oy1)

    def addr(u):
        t = u // NPT
        return t * TPL + (u - t * NPT) * B

    def p1(xbuf, ybuf, ibuf, wxbuf, wybuf):
        def f(gi, carry):
            x = xbuf[pl.ds(gi * L, L)]
            y = ybuf[pl.ds(gi * L, L)]
            xs = x * jnp.float32(W - 1)
            ys = y * jnp.float32(H - 1)
            x0 = xs.astype(jnp.int32)
            y0 = ys.astype(jnp.int32)
            ibuf[pl.ds(gi * L, L)] = y0 * W + x0
            wxbuf[pl.ds(gi * L, L)] = xs - x0.astype(jnp.float32)
            wybuf[pl.ds(gi * L, L)] = ys - y0.astype(jnp.float32)
            return carry

        lax.fori_loop(0, NG, f, 0, unroll=4)

    def p3(gbuf, wxbuf, wybuf, oxbuf, oybuf):
        def f(gi, carry):
            bse = gi * L
            rows = ids + bse
            gv = [plsc.load_gather(gbuf, [rows, jnp.full((L,), k, jnp.int32)])
                  for k in range(8)]
            wx = wxbuf[pl.ds(bse, L)]
            wy = wybuf[pl.ds(bse, L)]
            top0 = gv[0] + wx * (gv[2] - gv[0])
            top1 = gv[1] + wx * (gv[3] - gv[1])
            bot0 = gv[4] + wx * (gv[6] - gv[4])
            bot1 = gv[5] + wx * (gv[7] - gv[5])
            oxbuf[pl.ds(bse, L)] = top0 + wy * (bot0 - top0)
            oybuf[pl.ds(bse, L)] = top1 + wy * (bot1 - top1)
            return carry

        lax.fori_loop(0, NG, f, 0, unroll=4)

    b0 = addr(u0)
    pltpu.async_copy(coords_hbm.at[pl.ds(b0, B)], xb[0], csem)
    pltpu.async_copy(coords_hbm.at[pl.ds(b0 + N1, B)], yb[0], csem)

    def iteration(k, cur):
        nxt = 1 - cur
        base = addr(u0 + k)

        @pl.when(k + 1 < CPW)
        def _():
            bn = addr(u0 + k + 1)
            pltpu.async_copy(coords_hbm.at[pl.ds(bn, B)], xb[nxt], csem)
            pltpu.async_copy(coords_hbm.at[pl.ds(bn + N1, B)], yb[nxt], csem)

        pltpu.make_async_copy(coords_hbm.at[pl.ds(base, B)], xb[cur], csem).wait()
        pltpu.make_async_copy(coords_hbm.at[pl.ds(base + N1, B)], yb[cur], csem).wait()

        p1(xb[cur], yb[cur], ib[cur], wxb[cur], wyb[cur])
        pltpu.async_copy(grid8_hbm.at[ib[cur]], gb[cur], gsem)

        @pl.when(k >= 1)
        def _():
            @pl.when(k >= 2)
            def _():
                bq = addr(u0 + k - 2)
                pltpu.make_async_copy(oxb[cur], out_hbm.at[pl.ds(bq, B)], osem).wait()
                pltpu.make_async_copy(oyb[cur], out_hbm.at[pl.ds(bq + N1, B)], osem).wait()
            p3(gb[nxt], wxb[nxt], wyb[nxt], oxb[nxt], oyb[nxt])
            bp = addr(u0 + k - 1)
            pltpu.async_copy(oxb[nxt], out_hbm.at[pl.ds(bp, B)], osem)
            pltpu.async_copy(oyb[nxt], out_hbm.at[pl.ds(bp + N1, B)], osem)

        pltpu.make_async_copy(grid8_hbm.at[ib[cur]], gb[cur], gsem).wait()

    def two(j2, carry):
        iteration(j2 * 2, 0)
        iteration(j2 * 2 + 1, 1)
        return carry

    lax.fori_loop(0, CPW // 2, two, 0)

    lastp = (CPW - 1) % 2
    p3(gb[lastp], wxb[lastp], wyb[lastp], oxb[lastp], oyb[lastp])
    bl = addr(u0 + CPW - 1)
    pltpu.async_copy(oxb[lastp], out_hbm.at[pl.ds(bl, B)], osem)
    pltpu.async_copy(oyb[lastp], out_hbm.at[pl.ds(bl + N1, B)], osem)
    for _i in range(4):
        pltpu.make_async_copy(oxb[0], out_hbm.at[pl.ds(bl, B)], osem).wait()


def _sample(coords_planar, grid8):
    mesh = plsc.VectorSubcoreMesh(core_axis_name="c", subcore_axis_name="s")
    return pl.kernel(
        _sc_body,
        out_type=jax.ShapeDtypeStruct((T * 2 * N1,), jnp.float32),
        mesh=mesh,
        compiler_params=pltpu.CompilerParams(
            needs_layout_passes=False, use_tc_tiling_on_sc=False),
        scratch_types=(
            [pltpu.VMEM((B,), jnp.float32)] * 4
            + [pltpu.VMEM((B,), jnp.int32)] * 2
            + [pltpu.VMEM((B,), jnp.float32)] * 4
            + [pltpu.VMEM((B, 8), jnp.float32)] * 2
            + [pltpu.VMEM((B,), jnp.float32)] * 4
            + [pltpu.SemaphoreType.DMA] * 3
        ),
    )(coords_planar, grid8)


def kernel(coords, vector_field):
    coords_planar = jnp.transpose(coords, (1, 2, 0)).reshape(-1)
    grid_planar = jnp.transpose(vector_field, (0, 2, 1)).reshape(-1)
    grid8 = _build_grid8(grid_planar).reshape(H * W, 8)
    out_flat = _sample(coords_planar, grid8)
    return jnp.transpose(out_flat.reshape(T, 2, N1), (2, 0, 1))
